# baseline (speedup 1.0000x reference)
import functools

import jax
import jax.numpy as jnp
from jax import lax
from jax.experimental import pallas as pl
from jax.experimental.pallas import tpu as pltpu
from jax.experimental.pallas import tpu_sc as plsc

_NC = 2
_NS = 16
_NW = _NC * _NS
_L = 16

_R_SC = 4352


def _sc_body(x_hbm, tab_hbm, o_hbm, tabv, buf, *, row_lo, per_w, ch, b):
    c = lax.axis_index("c")
    s = lax.axis_index("s")
    wid = s * _NC + c
    row0 = row_lo + wid * per_w
    nch = per_w // ch
    nvec = b // _L
    pltpu.sync_copy(tab_hbm.at[pl.ds(row0, per_w)], tabv)

    def chunk(j, carry):
        base = row0 + j * ch
        pltpu.sync_copy(x_hbm.at[pl.ds(base, ch)], buf)
        for srow in range(ch):
            tsp = jnp.reshape(tabv[pl.ds(j * ch + srow, 1), :], (_L,))

            def vec(k, carry2):
                for u in range(8):
                    off = (k * 8 + u) * _L
                    buf[srow, pl.ds(off, _L)] = buf[srow, pl.ds(off, _L)] + tsp
                return carry2

            lax.fori_loop(0, nvec // 8, vec, 0)
        pltpu.sync_copy(buf, o_hbm.at[pl.ds(base - row_lo, ch)])
        return carry

    lax.fori_loop(0, nch, chunk, 0)


def _sc_add(x, tab16, row_lo, rows):
    R, B = x.shape
    per_w = rows // _NW
    ch = 8
    mesh = plsc.VectorSubcoreMesh(core_axis_name="c", subcore_axis_name="s")
    return pl.kernel(
        functools.partial(_sc_body, row_lo=row_lo, per_w=per_w, ch=ch, b=B),
        out_type=jax.ShapeDtypeStruct((rows, B), jnp.float32),
        mesh=mesh,
        scratch_types=[
            pltpu.VMEM((per_w, _L), jnp.float32),
            pltpu.VMEM((ch, B), jnp.float32),
        ],
    )(x, tab16)


def _tc_body(x_ref, t_ref, o_ref):
    o_ref[...] = x_ref[...] + t_ref[:, 0:1]


def _tc_add(x, tab16, rows):
    R, B = x.shape
    BR = 64
    grid = (rows // BR,)
    return pl.pallas_call(
        _tc_body,
        grid=grid,
        in_specs=[
            pl.BlockSpec((BR, B), lambda i: (i, 0)),
            pl.BlockSpec((BR, _L), lambda i: (i, 0)),
        ],
        out_specs=pl.BlockSpec((BR, B), lambda i: (i, 0)),
        out_shape=jax.ShapeDtypeStruct((rows, B), jnp.float32),
        compiler_params=pltpu.CompilerParams(
            dimension_semantics=("arbitrary",),
        ),
    )(x[:rows], tab16[:rows])


def kernel(inputs, lookup_table):
    B, T, U = inputs.shape
    scale = float(U) ** 0.5
    R = T * U
    x = jnp.transpose(inputs, (1, 2, 0)).reshape(R, B)
    tab16 = jnp.broadcast_to(
        (lookup_table.reshape(R) * scale)[:, None], (R, _L)
    )
    r_tc = R - _R_SC
    out_sc = _sc_add(x, tab16, r_tc, _R_SC)
    out_tc = _tc_add(x, tab16, r_tc)
    out = jnp.concatenate([out_tc, out_sc], axis=0)
    return jnp.transpose(out.reshape(T, U, B), (2, 0, 1))


# speedup vs baseline: 3.2758x; 3.2758x over previous
import functools

import jax
import jax.numpy as jnp
from jax.experimental import pallas as pl
from jax.experimental.pallas import tpu as pltpu


def _body(x_ref, t_ref, o_ref, *, scale):
    t = t_ref[...] * scale
    o_ref[...] = x_ref[...] + t[:, :, None]


def kernel(inputs, lookup_table):
    B, T, U = inputs.shape
    scale = float(U) ** 0.5

    x = jnp.transpose(inputs, (1, 2, 0))

    BT = 8
    grid = (T // BT,)
    out = pl.pallas_call(
        functools.partial(_body, scale=scale),
        grid=grid,
        in_specs=[
            pl.BlockSpec((BT, U, B), lambda i: (i, 0, 0)),
            pl.BlockSpec((BT, U), lambda i: (i, 0)),
        ],
        out_specs=pl.BlockSpec((BT, U, B), lambda i: (i, 0, 0)),
        out_shape=jax.ShapeDtypeStruct((T, U, B), jnp.float32),
        compiler_params=pltpu.CompilerParams(
            dimension_semantics=("arbitrary",),
        ),
    )(x, lookup_table)
    return jnp.transpose(out, (2, 0, 1))


# speedup vs baseline: 3.3132x; 1.0114x over previous
import functools

import jax
import jax.numpy as jnp
from jax.experimental import pallas as pl
from jax.experimental.pallas import tpu as pltpu


def _body(x_ref, t_ref, o_ref, ts_ref, *, scale, bt):
    i = pl.program_id(0)

    @pl.when(i == 0)
    def _():
        ts_ref[...] = jnp.transpose(t_ref[...], (1, 0)) * scale

    t = ts_ref[pl.ds(pl.multiple_of(i * bt, bt), bt), :]
    o_ref[...] = x_ref[...] + t[:, :, None]


def kernel(inputs, lookup_table):
    B, T, U = inputs.shape
    scale = float(U) ** 0.5

    x = jnp.transpose(inputs, (1, 2, 0))
    tab_t = jnp.transpose(lookup_table, (1, 0))

    BT = 8
    grid = (T // BT,)
    out = pl.pallas_call(
        functools.partial(_body, scale=scale, bt=BT),
        grid=grid,
        in_specs=[
            pl.BlockSpec((BT, U, B), lambda i: (i, 0, 0)),
            pl.BlockSpec((U, T), lambda i: (0, 0)),
        ],
        out_specs=pl.BlockSpec((BT, U, B), lambda i: (i, 0, 0)),
        out_shape=jax.ShapeDtypeStruct((T, U, B), jnp.float32),
        scratch_shapes=[pltpu.VMEM((T, U), jnp.float32)],
        compiler_params=pltpu.CompilerParams(
            dimension_semantics=("arbitrary",),
        ),
    )(x, tab_t)
    return jnp.transpose(out, (2, 0, 1))
